# baseline (speedup 1.0000x reference)
import jax
import jax.numpy as jnp
from jax.experimental import pallas as pl
from jax.experimental.pallas import tpu as pltpu

N_NODES_ = 10000
D_FEAT_ = 128
PE_DIM_ = 256
PE_EMBED_DIM_ = 512
OUT_D_ = D_FEAT_ + PE_EMBED_DIM_

SIZES = (4000, 6000)
OFFS = (0, 4000)
G = len(SIZES)


def _manual_kernel(x_hbm, pe_hbm, w_ref, b_ref, out_hbm,
                   x_buf, pe_buf, stage, sem_x, sem_pe, sem_out):
    def x_in(i):
        o, m = OFFS[i], SIZES[i]
        return pltpu.make_async_copy(
            x_hbm.at[pl.ds(o, m), :], x_buf.at[pl.ds(o, m), :], sem_x.at[i])

    def pe_in(i):
        o, m = OFFS[i], SIZES[i]
        return pltpu.make_async_copy(
            pe_hbm.at[pl.ds(o, m), :], pe_buf.at[pl.ds(o, m), :], sem_pe.at[i])

    def out_cp(i):
        o, m = OFFS[i], SIZES[i]
        return pltpu.make_async_copy(
            stage.at[pl.ds(o, m), :], out_hbm.at[pl.ds(o, m), :], sem_out.at[i])

    for i in range(G):
        x_in(i).start()
        pe_in(i).start()
    for i in range(G):
        o, m = OFFS[i], SIZES[i]
        pe_in(i).wait()
        acc = jnp.dot(pe_buf[pl.ds(o, m), :], w_ref[:],
                      preferred_element_type=jnp.float32)
        stage[pl.ds(o, m), D_FEAT_:] = acc + b_ref[:]
        x_in(i).wait()
        stage[pl.ds(o, m), :D_FEAT_] = x_buf[pl.ds(o, m), :]
        out_cp(i).start()
    for i in range(G):
        out_cp(i).wait()


def kernel(x, e_index, pos_enc, W, b):
    del e_index
    n = x.shape[0]
    out = pl.pallas_call(
        _manual_kernel,
        in_specs=[
            pl.BlockSpec(memory_space=pltpu.MemorySpace.HBM),
            pl.BlockSpec(memory_space=pltpu.MemorySpace.HBM),
            pl.BlockSpec(memory_space=pltpu.MemorySpace.VMEM),
            pl.BlockSpec(memory_space=pltpu.MemorySpace.VMEM),
        ],
        out_specs=pl.BlockSpec(memory_space=pltpu.MemorySpace.HBM),
        out_shape=jax.ShapeDtypeStruct((n, OUT_D_), jnp.float32),
        scratch_shapes=[
            pltpu.VMEM((N_NODES_, D_FEAT_), jnp.float32),
            pltpu.VMEM((N_NODES_, PE_DIM_), jnp.float32),
            pltpu.VMEM((N_NODES_, OUT_D_), jnp.float32),
            pltpu.SemaphoreType.DMA((G,)),
            pltpu.SemaphoreType.DMA((G,)),
            pltpu.SemaphoreType.DMA((G,)),
        ],
    )(x, pos_enc, W, b)
    return out


# speedup vs baseline: 1.0762x; 1.0762x over previous
import jax
import jax.numpy as jnp
from jax.experimental import pallas as pl
from jax.experimental.pallas import tpu as pltpu

N_NODES_ = 10000
D_FEAT_ = 128
PE_DIM_ = 256
PE_EMBED_DIM_ = 512
OUT_D_ = D_FEAT_ + PE_EMBED_DIM_

BLK = 1000
G = N_NODES_ // BLK


def _manual_kernel(x_hbm, pe_hbm, w_ref, b_ref, out_hbm,
                   x_buf, pe_buf, stage, sem_x, sem_pe, sem_out):
    def x_in(i):
        o = i * BLK
        return pltpu.make_async_copy(
            x_hbm.at[pl.ds(o, BLK), :], x_buf.at[pl.ds(o, BLK), :], sem_x.at[i])

    def pe_in(i):
        o = i * BLK
        return pltpu.make_async_copy(
            pe_hbm.at[pl.ds(o, BLK), :], pe_buf.at[pl.ds(o, BLK), :], sem_pe.at[i])

    def out_cp(i):
        o = i * BLK
        return pltpu.make_async_copy(
            stage.at[pl.ds(o, BLK), :], out_hbm.at[pl.ds(o, BLK), :], sem_out.at[i])

    for i in range(G):
        pe_in(i).start()
        x_in(i).start()
    for i in range(G):
        o = i * BLK
        pe_in(i).wait()
        acc = jnp.dot(pe_buf[pl.ds(o, BLK), :], w_ref[:],
                      preferred_element_type=jnp.float32)
        stage[pl.ds(o, BLK), D_FEAT_:] = acc + b_ref[:]
        x_in(i).wait()
        stage[pl.ds(o, BLK), :D_FEAT_] = x_buf[pl.ds(o, BLK), :]
        out_cp(i).start()
    for i in range(G):
        out_cp(i).wait()


def kernel(x, e_index, pos_enc, W, b):
    del e_index
    n = x.shape[0]
    out = pl.pallas_call(
        _manual_kernel,
        in_specs=[
            pl.BlockSpec(memory_space=pltpu.MemorySpace.HBM),
            pl.BlockSpec(memory_space=pltpu.MemorySpace.HBM),
            pl.BlockSpec(memory_space=pltpu.MemorySpace.VMEM),
            pl.BlockSpec(memory_space=pltpu.MemorySpace.VMEM),
        ],
        out_specs=pl.BlockSpec(memory_space=pltpu.MemorySpace.HBM),
        out_shape=jax.ShapeDtypeStruct((n, OUT_D_), jnp.float32),
        scratch_shapes=[
            pltpu.VMEM((N_NODES_, D_FEAT_), jnp.float32),
            pltpu.VMEM((N_NODES_, PE_DIM_), jnp.float32),
            pltpu.VMEM((N_NODES_, OUT_D_), jnp.float32),
            pltpu.SemaphoreType.DMA((G,)),
            pltpu.SemaphoreType.DMA((G,)),
            pltpu.SemaphoreType.DMA((G,)),
        ],
    )(x, pos_enc, W, b)
    return out
